# Initial kernel scaffold; baseline (speedup 1.0000x reference)
import jax
import jax.numpy as jnp
from jax.experimental import pallas as pl


def kernel(input, index, value):
    raise NotImplementedError("write your pallas kernel here")


# speedup vs baseline: 1.5085x; 1.5085x over previous
import jax
import jax.numpy as jnp
import numpy as np

_Z = np.int32(0)
from jax.experimental import pallas as pl
from jax.experimental.pallas import tpu as pltpu

_N_ROWS = 100000
_N_UPD = 16384
_ROW32 = 256
_COPY_BLK = 2000


def _copy_body(x_ref, o_ref):
    o_ref[...] = x_ref[...]


def _scatter_body(idx_ref, v_ref, o_alias_ref, o_ref):
    del idx_ref, o_alias_ref
    o_ref[...] = v_ref[...]


def kernel(input, index, value):
    x = jax.lax.bitcast_convert_type(input, jnp.int32).reshape(_N_ROWS, _ROW32)
    v = jax.lax.bitcast_convert_type(value, jnp.int32).reshape(_N_UPD, _ROW32)
    idx = index.astype(jnp.int32)

    out0 = pl.pallas_call(
        _copy_body,
        grid=(_N_ROWS // _COPY_BLK,),
        in_specs=[pl.BlockSpec((_COPY_BLK, _ROW32), lambda i: (i, _Z))],
        out_specs=pl.BlockSpec((_COPY_BLK, _ROW32), lambda i: (i, _Z)),
        out_shape=jax.ShapeDtypeStruct((_N_ROWS, _ROW32), jnp.int32),
    )(x)

    grid_spec = pltpu.PrefetchScalarGridSpec(
        num_scalar_prefetch=1,
        grid=(_N_UPD,),
        in_specs=[
            pl.BlockSpec((1, 1, _ROW32), lambda i, idx_ref: (i, _Z, _Z)),
            pl.BlockSpec((1, 1, _ROW32), lambda i, idx_ref: (idx_ref[i], _Z, _Z)),
        ],
        out_specs=pl.BlockSpec((1, 1, _ROW32), lambda i, idx_ref: (idx_ref[i], _Z, _Z)),
    )
    out1 = pl.pallas_call(
        _scatter_body,
        grid_spec=grid_spec,
        out_shape=jax.ShapeDtypeStruct((_N_ROWS, 1, _ROW32), jnp.int32),
        input_output_aliases={2: 0},
    )(idx, v.reshape(_N_UPD, 1, _ROW32), out0.reshape(_N_ROWS, 1, _ROW32))

    out = out1.reshape(_N_ROWS, 16, 8, 2)
    return jax.lax.bitcast_convert_type(out, jnp.int64)
